# baseline (speedup 1.0000x reference)
import functools

import jax
import jax.numpy as jnp
from jax import lax
from jax.experimental import pallas as pl
from jax.experimental.pallas import tpu as pltpu
from jax.experimental.pallas import tpu_sc as plsc

N = 10000
E = 320000
D = 128
TILES = 16
GRP = 128
K = 8
CH = 20
G = CH * K
EPT = G * GRP
EPAD = TILES * EPT
RPT = 632
NP = TILES * RPT
_OUT_CHUNKS = ((0, GRP), (1, GRP), (2, GRP), (3, GRP), (4, RPT - 4 * GRP))


GG = 64
KG = 8
KS = 8
CHB = 40

_PERM = []
for _j in range(D // 32):
    _PERM += [32 * _j + 2 * _p for _p in range(16)]
    _PERM += [32 * _j + 2 * _p + 1 for _p in range(16)]


def _sc_segment_sum_bf16(feats16, srcs, dsts, z128):
    mesh = plsc.VectorSubcoreMesh(core_axis_name="c", subcore_axis_name="s")

    @functools.partial(
        pl.kernel,
        mesh=mesh,
        out_type=jax.ShapeDtypeStruct((2, NP, D), jnp.float32),
        compiler_params=pltpu.CompilerParams(use_tc_tiling_on_sc=False,
                                             needs_layout_passes=False),
        scratch_types=[
            pltpu.VMEM((KG, GG), jnp.int32),
            pltpu.VMEM((KG, GG), jnp.int32),
            pltpu.VMEM((KS, GG), jnp.int32),
            pltpu.VMEM((KS, GG), jnp.int32),
            pltpu.VMEM((GG, D // 2), jnp.int32),
            pltpu.VMEM((GG, D // 2), jnp.int32),
            pltpu.VMEM((GG, D // 2), jnp.int32),
            pltpu.VMEM((GRP, D), jnp.float32),
            pltpu.SemaphoreType.DMA,
            pltpu.SemaphoreType.DMA,
            pltpu.SemaphoreType.DMA,
            pltpu.SemaphoreType.DMA,
            pltpu.SemaphoreType.DMA,
            pltpu.SemaphoreType.DMA,
            pltpu.SemaphoreType.DMA,
            pltpu.VMEM_SHARED((NP, D), jnp.float32),
        ],
    )
    def k(feats_h, srcs_h, dsts_h, z128_h,
          sums_o, idx_s0, idx_s1, idx_d0, idx_d1, gb0, gb1, gb2, fb,
          sg0, sg1, sg2, ss0, ss1, si0, si1, acc):
        c = lax.axis_index("c")
        s = lax.axis_index("s")
        row0 = s * RPT
        gbs = (gb0, gb1, gb2)
        gsem = (sg0, sg1, sg2)
        pltpu.sync_copy(z128_h, fb)
        for i, w in _OUT_CHUNKS:
            pltpu.sync_copy(fb.at[pl.ds(0, w)],
                            acc.at[pl.ds(row0 + i * GRP, w)])
        plsc.subcore_barrier()

        def convert(gb, base):
            def cv(r2, carry):
                for dr in (0, 1):
                    r = r2 * 2 + dr
                    for j in range(D // 32):
                        v = plsc.bitcast(gb[r, pl.ds(j * 16, 16)],
                                         jnp.bfloat16)
                        a, b = plsc.unpack(
                            v, format=plsc.PackFormat.INTERLEAVED)
                        fb[base + r, pl.ds(j * 32, 16)] = a
                        fb[base + r, pl.ds(j * 32 + 16, 16)] = b
                return carry

            lax.fori_loop(0, GG // 2, cv, 0)

        isx = (idx_s0, idx_s1)
        idx = (idx_d0, idx_d1)
        isem = (si0, si1)
        pltpu.async_copy(srcs_h.at[c, s, 0], idx_s0, si0)
        pltpu.async_copy(dsts_h.at[c, s, 0], idx_d0, si0)
        pltpu.async_copy(srcs_h.at[c, s, 1], idx_s1, si1)
        pltpu.async_copy(dsts_h.at[c, s, 1], idx_d1, si1)

        def pair(p, carry):
            for b in (0, 1):
                ch = 2 * p + b
                idx_s = isx[b]
                idx_d = idx[b]
                pltpu.make_async_copy(srcs_h.at[c, s, ch], idx_s,
                                      isem[b]).wait()
                pltpu.make_async_copy(dsts_h.at[c, s, ch], idx_d,
                                      isem[b]).wait()
                dgq = [None] * (KG + 3)
                for g0 in (0, 1, 2):
                    dgq[g0] = pltpu.async_copy(
                        feats_h.at[idx_s.at[g0]], gbs[g0], gsem[g0])
                ssem = (ss0, ss1)
                dsc = {}
                for g in range(KG):
                    h = g % 2
                    dgq[g].wait()
                    if g >= 2:
                        dsc[g - 2].wait()
                    convert(gbs[g % 3], h * GG)
                    if g + 3 < KG:
                        dgq[g + 3] = pltpu.async_copy(
                            feats_h.at[idx_s.at[g + 3]], gbs[g % 3],
                            gsem[g % 3])
                    if g == KG - 1:
                        ch2 = jnp.minimum(ch + 2, CHB - 1)
                        pltpu.async_copy(srcs_h.at[c, s, ch2], idx_s,
                                         isem[b])
                        pltpu.async_copy(dsts_h.at[c, s, ch2], idx_d,
                                         isem[b])
                    dsc[g] = pltpu.async_copy(
                        fb.at[pl.ds(h * GG, GG)], acc.at[idx_d.at[g]],
                        ssem[h], add=True)
                dsc[KG - 2].wait()
                dsc[KG - 1].wait()
            return carry

        lax.fori_loop(0, CHB // 2, pair, 0)
        for b in (0, 1):
            pltpu.make_async_copy(srcs_h.at[c, s, CHB - 1], isx[b],
                                  isem[b]).wait()
            pltpu.make_async_copy(dsts_h.at[c, s, CHB - 1], idx[b],
                                  isem[b]).wait()
        plsc.subcore_barrier()
        for i, w in _OUT_CHUNKS:
            pltpu.sync_copy(acc.at[pl.ds(row0 + i * GRP, w)],
                            fb.at[pl.ds(0, w)])
            pltpu.sync_copy(fb.at[pl.ds(0, w)],
                            sums_o.at[c, pl.ds(row0 + i * GRP, w)])

    return k(feats16, srcs, dsts, z128)


DC = 64


def _sc_count(dsts, z64, ones_h):
    mesh = plsc.VectorSubcoreMesh(core_axis_name="c", subcore_axis_name="s")

    @functools.partial(
        pl.kernel,
        mesh=mesh,
        out_type=jax.ShapeDtypeStruct((2, NP, DC), jnp.float32),
        compiler_params=pltpu.CompilerParams(use_tc_tiling_on_sc=False,
                                             needs_layout_passes=False),
        scratch_types=[
            pltpu.VMEM((K, GRP), jnp.int32),
            pltpu.VMEM((GRP, DC), jnp.float32),
            pltpu.SemaphoreType.DMA,
            pltpu.VMEM_SHARED((NP, DC), jnp.float32),
        ],
    )
    def k(dsts_h, z64_h, ones_hbm, cnt_o, idx_d, rows, sem, acc):
        c = lax.axis_index("c")
        s = lax.axis_index("s")
        row0 = s * RPT
        pltpu.sync_copy(z64_h, rows)
        for i, w in _OUT_CHUNKS:
            pltpu.sync_copy(rows.at[pl.ds(0, w)],
                            acc.at[pl.ds(row0 + i * GRP, w)])
        pltpu.sync_copy(ones_hbm, rows)
        plsc.subcore_barrier()

        def chunk(ch, carry):
            pltpu.sync_copy(dsts_h.at[c, s, ch], idx_d)
            dsc = [pltpu.async_copy(rows, acc.at[idx_d.at[g]], sem,
                                    add=True)
                   for g in range(K)]
            for d in dsc:
                d.wait()
            return carry

        lax.fori_loop(0, CH, chunk, 0)
        plsc.subcore_barrier()
        for i, w in _OUT_CHUNKS:
            pltpu.sync_copy(acc.at[pl.ds(row0 + i * GRP, w)], rows.at[pl.ds(0, w)])
            pltpu.sync_copy(rows.at[pl.ds(0, w)],
                            cnt_o.at[c, pl.ds(row0 + i * GRP, w)])

    return k(dsts, z64, ones_h)


def _combine(x, sums, degs, ws0, ws1, wn0, wn1, b0, b1, relu):
    blk = 1000

    def body(x_ref, s0_ref, s1_ref, d0_ref, d1_ref,
             ws0_ref, ws1_ref, wn0_ref, wn1_ref, b_ref, o_ref):
        xb = x_ref[...]
        r0 = 1.0 / jnp.maximum(d0_ref[0][:, 0:1], 1.0)
        r1 = 1.0 / jnp.maximum(d1_ref[0][:, 0:1], 1.0)
        acc = jnp.dot(xb, ws0_ref[...] + ws1_ref[...],
                      preferred_element_type=jnp.float32)
        acc = acc + jnp.dot(s0_ref[0] * r0, wn0_ref[...],
                            preferred_element_type=jnp.float32)
        acc = acc + jnp.dot(s1_ref[0] * r1, wn1_ref[...],
                            preferred_element_type=jnp.float32)
        acc = acc + b_ref[...]
        if relu:
            acc = jnp.maximum(acc, 0.0)
        o_ref[...] = acc

    row_spec = pl.BlockSpec((blk, D), lambda i: (i, 0))
    sum0_spec = pl.BlockSpec((1, blk, D), lambda i: (0, i, 0))
    sum1_spec = pl.BlockSpec((1, blk, D), lambda i: (1, i, 0))
    deg0_spec = pl.BlockSpec((1, blk, DC), lambda i: (0, i, 0))
    deg1_spec = pl.BlockSpec((1, blk, DC), lambda i: (1, i, 0))
    w_spec = pl.BlockSpec((D, D), lambda i: (0, 0))
    b_spec = pl.BlockSpec((1, D), lambda i: (0, 0))
    bias = (b0 + b1).reshape(1, D)
    return pl.pallas_call(
        body,
        grid=(N // blk,),
        in_specs=[row_spec, sum0_spec, sum1_spec, deg0_spec, deg1_spec,
                  w_spec, w_spec, w_spec, w_spec, b_spec],
        out_specs=row_spec,
        out_shape=jax.ShapeDtypeStruct((N, D), jnp.float32),
    )(x, sums, sums, degs, degs, ws0, ws1, wn0, wn1, bias)


def _as_i32view(arr):
    a16 = arr.astype(jnp.bfloat16)
    return jax.lax.bitcast_convert_type(
        a16.reshape(arr.shape[0], D // 2, 2), jnp.int32)


def _prep_edges(ei0, ei1):
    pad = EPAD - E
    parts = []
    for ei in (ei0, ei1):
        src = jnp.concatenate([ei[0], jnp.zeros((pad,), jnp.int32)])
        dst = jnp.concatenate([ei[1], jnp.full((pad,), N, jnp.int32)])
        parts.append((src.reshape(TILES, EPT), dst.reshape(TILES, EPT)))
    srcs = jnp.stack([parts[0][0], parts[1][0]])
    dsts = jnp.stack([parts[0][1], parts[1][1]])
    return srcs, dsts


def kernel(x, edge_index_rel0, edge_index_rel1,
           l1_rel0_Wself, l1_rel0_Wneigh, l1_rel0_b,
           l1_rel1_Wself, l1_rel1_Wneigh, l1_rel1_b,
           l2_rel0_Wself, l2_rel0_Wneigh, l2_rel0_b,
           l2_rel1_Wself, l2_rel1_Wneigh, l2_rel1_b):
    srcs, dsts = _prep_edges(edge_index_rel0, edge_index_rel1)
    dsts_cnt = dsts.reshape(2, TILES, CH, K, GRP)
    srcs_b = srcs.reshape(2, TILES, CHB, KG, GG)
    dsts_b = dsts.reshape(2, TILES, CHB, KS, GG)
    z128 = jnp.zeros((GRP, D), jnp.float32)

    ones_h = jnp.ones((GRP, DC), jnp.float32)
    z64 = jnp.zeros((GRP, DC), jnp.float32)
    degs = _sc_count(dsts_cnt, z64, ones_h)

    perm = jnp.asarray(_PERM)
    x16 = _as_i32view(x)
    sums1 = _sc_segment_sum_bf16(x16, srcs_b, dsts_b, z128)
    h = _combine(x, sums1, degs,
                 l1_rel0_Wself, l1_rel1_Wself,
                 l1_rel0_Wneigh[perm], l1_rel1_Wneigh[perm],
                 l1_rel0_b, l1_rel1_b, relu=True)
    h16 = _as_i32view(h)
    sums2 = _sc_segment_sum_bf16(h16, srcs_b, dsts_b, z128)
    out = _combine(h, sums2, degs,
                   l2_rel0_Wself, l2_rel1_Wself,
                   l2_rel0_Wneigh[perm], l2_rel1_Wneigh[perm],
                   l2_rel0_b, l2_rel1_b, relu=False)
    return out
